# baseline (speedup 1.0000x reference)
import jax
import jax.numpy as jnp
from jax.experimental import pallas as pl
from jax.experimental.pallas import tpu as pltpu

B = 16
N = 32768
IN = 128
HID = 256
MH = HID // 2
OUT = 64
RB = 2048
NB = N // RB
NEG = -1e30
EPS = 1e-5


def _seg_partial_max(bmin, bmax, bcol, x, ref):
    @pl.when(bmin == bmax)
    def _():
        m = jnp.max(x, axis=0, keepdims=True)
        ref[pl.ds(bmin, 1), :] = jnp.maximum(ref[pl.ds(bmin, 1), :], m)

    @pl.when(bmin != bmax)
    def _():
        for s in range(B):
            @pl.when((bmin <= s) & (s <= bmax))
            def _(s=s):
                m = jnp.max(jnp.where(bcol == s, x, NEG), axis=0,
                            keepdims=True)
                ref[s:s + 1, :] = jnp.maximum(ref[s:s + 1, :], m)


def _dot_t(x, w):
    return jax.lax.dot_general(
        x, w, (((1,), (1,)), ((), ())), preferred_element_type=jnp.float32
    )


def _dot(x, w):
    return jax.lax.dot_general(
        x, w, (((1,), (0,)), ((), ())), preferred_element_type=jnp.float32
    )


def _ln_prelu(h, nw, nb, a):
    mu = jnp.mean(h, axis=1, keepdims=True)
    ms = jnp.mean(h * h, axis=1, keepdims=True)
    var = jnp.maximum(ms - mu * mu, 0.0)
    y = (h - mu) * jax.lax.rsqrt(var + EPS) * nw + nb
    return jnp.where(y >= 0, y, a * y)


def _onehot(bcol):
    return (bcol == jax.lax.broadcasted_iota(
        jnp.int32, (bcol.shape[0], B), 1)).astype(jnp.float32)


def _deepset_kernel(bnds_ref, pos_ref, b_ref, cnt_ref, l1w_ref, g1w_ref,
                    g1b_ref, n1w_ref, n1b_ref, a1_ref, l2w_ref, g2w_ref,
                    g2b_ref, n2w_ref, n2b_ref, a2_ref, m1w_ref, m1b_ref,
                    mnw_ref, mnb_ref, ma_ref, m2w_ref, m2b_ref, out_ref,
                    smax0_s, x1_s, smax1_s, ssum_s):
    i = pl.program_id(0)

    @pl.when(i == 0)
    def _():
        smax0_s[...] = jnp.full((B, IN), NEG, jnp.float32)
        smax1_s[...] = jnp.full((B, HID), NEG, jnp.float32)
        ssum_s[...] = jnp.zeros((B, HID), jnp.float32)

    bcol = b_ref[0]
    j = i % NB
    bmin = bnds_ref[j, 0]
    bmax = bnds_ref[j, 1]
    row = pl.multiple_of(j * RB, RB)

    @pl.when(i < NB)
    def _phase_a():
        _seg_partial_max(bmin, bmax, bcol, pos_ref[...], smax0_s)

    @pl.when((i >= NB) & (i < 2 * NB))
    def _phase_b():
        xm = _dot_t(smax0_s[...], l1w_ref[...])
        gath = _dot(_onehot(bcol), xm)
        h = _dot_t(pos_ref[...], g1w_ref[...]) + g1b_ref[...] - gath
        y = _ln_prelu(h, n1w_ref[...], n1b_ref[...], a1_ref[0, 0])
        x1_s[pl.ds(row, RB), :] = y
        _seg_partial_max(bmin, bmax, bcol, y, smax1_s)

    @pl.when(i >= 2 * NB)
    def _phase_c():
        mask = _onehot(bcol)
        xm = _dot_t(smax1_s[...], l2w_ref[...])
        gath = _dot(mask, xm)
        x = x1_s[pl.ds(row, RB), :]
        h = _dot_t(x, g2w_ref[...]) + g2b_ref[...] - gath
        y = _ln_prelu(h, n2w_ref[...], n2b_ref[...], a2_ref[0, 0])
        ssum_s[...] += jax.lax.dot_general(
            mask, y, (((0,), (0,)), ((), ())),
            preferred_element_type=jnp.float32)

        @pl.when(i == 3 * NB - 1)
        def _head():
            eye = (jax.lax.broadcasted_iota(jnp.int32, (B, B), 0)
                   == jax.lax.broadcasted_iota(jnp.int32, (B, B), 1)
                   ).astype(jnp.float32)
            cnt_col = _dot_t(eye, cnt_ref[...])
            pooled = ssum_s[...] / jnp.maximum(cnt_col, 1.0)
            hh = _dot_t(pooled, m1w_ref[...]) + m1b_ref[...]
            hh = _ln_prelu(hh, mnw_ref[...], mnb_ref[...], ma_ref[0, 0])
            out_ref[...] = _dot_t(hh, m2w_ref[...]) + m2b_ref[...]


def _row(v):
    return v.reshape(1, -1)


def kernel(pos, batch, g1_w, g1_b, l1_w, n1_w, n1_b, a1, g2_w, g2_b, l2_w,
           n2_w, n2_b, a2, m1_w, m1_b, mn_w, mn_b, ma, m2_w, m2_b):
    batch = batch.astype(jnp.int32)
    b3 = batch.reshape(NB, RB, 1)
    bnds = jnp.stack([batch[::RB], batch[RB - 1::RB]], axis=1)
    edges = jnp.searchsorted(batch, jnp.arange(B + 1, dtype=jnp.int32))
    cnts = (edges[1:] - edges[:-1]).astype(jnp.float32).reshape(1, B)
    a1r, a2r, mar = a1.reshape(1, 1), a2.reshape(1, 1), ma.reshape(1, 1)

    full = lambda a: pl.BlockSpec(a.shape, lambda i, b: (0,) * a.ndim)

    grid_spec = pltpu.PrefetchScalarGridSpec(
        num_scalar_prefetch=1,
        grid=(3 * NB,),
        in_specs=[
            pl.BlockSpec((RB, IN),
                         lambda i, b: (jnp.where(i < 2 * NB, i % NB, NB - 1),
                                       0)),
            pl.BlockSpec((1, RB, 1), lambda i, b: (i % NB, 0, 0)),
            full(cnts), full(l1_w), full(g1_w), full(_row(g1_b)), full(_row(n1_w)),
            full(_row(n1_b)), full(a1r),
            full(l2_w), full(g2_w), full(_row(g2_b)), full(_row(n2_w)),
            full(_row(n2_b)), full(a2r),
            full(m1_w), full(_row(m1_b)), full(_row(mn_w)), full(_row(mn_b)),
            full(mar), full(m2_w), full(_row(m2_b)),
        ],
        out_specs=pl.BlockSpec((B, OUT), lambda i, b: (0, 0)),
        scratch_shapes=[
            pltpu.VMEM((B, IN), jnp.float32),
            pltpu.VMEM((N, HID), jnp.float32),
            pltpu.VMEM((B, HID), jnp.float32),
            pltpu.VMEM((B, HID), jnp.float32),
        ],
    )

    out = pl.pallas_call(
        _deepset_kernel,
        grid_spec=grid_spec,
        out_shape=jax.ShapeDtypeStruct((B, OUT), jnp.float32),
    )(bnds, pos, b3, cnts, l1_w, g1_w, _row(g1_b), _row(n1_w), _row(n1_b),
      a1r, l2_w, g2_w, _row(g2_b), _row(n2_w), _row(n2_b), a2r,
      m1_w, _row(m1_b), _row(mn_w), _row(mn_b), mar, m2_w, _row(m2_b))
    return out


# speedup vs baseline: 1.1814x; 1.1814x over previous
import jax
import jax.numpy as jnp
from jax.experimental import pallas as pl
from jax.experimental.pallas import tpu as pltpu

B = 16
N = 32768
IN = 128
HID = 256
MH = HID // 2
OUT = 64
RB = 2048
NB = N // RB
NEG = -1e30
EPS = 1e-5


def _seg_partial_max(bmin, bmax, bcol, x, ref):
    @pl.when(bmin == bmax)
    def _():
        m = jnp.max(x, axis=0, keepdims=True)
        ref[pl.ds(bmin, 1), :] = jnp.maximum(ref[pl.ds(bmin, 1), :], m)

    @pl.when(bmin != bmax)
    def _():
        for s in range(B):
            @pl.when((bmin <= s) & (s <= bmax))
            def _(s=s):
                m = jnp.max(jnp.where(bcol == s, x, NEG), axis=0,
                            keepdims=True)
                ref[s:s + 1, :] = jnp.maximum(ref[s:s + 1, :], m)


def _dot_t(x, w):
    return jax.lax.dot_general(
        x, w, (((1,), (1,)), ((), ())), preferred_element_type=jnp.float32
    )


def _dot(x, w):
    return jax.lax.dot_general(
        x, w, (((1,), (0,)), ((), ())), preferred_element_type=jnp.float32
    )


def _ln_prelu(h, nw, nb, a):
    mu = jnp.mean(h, axis=1, keepdims=True)
    ms = jnp.mean(h * h, axis=1, keepdims=True)
    var = jnp.maximum(ms - mu * mu, 0.0)
    y = (h - mu) * jax.lax.rsqrt(var + EPS) * nw + nb
    return jnp.where(y >= 0, y, a * y)


def _onehot(bcol):
    return (bcol == jax.lax.broadcasted_iota(
        jnp.int32, (bcol.shape[0], B), 1)).astype(jnp.float32)


def _deepset_kernel(bnds_ref, pos_ref, b_ref, l1w_ref, g1w_ref,
                    g1b_ref, n1w_ref, n1b_ref, a1_ref, l2w_ref, g2w_ref,
                    g2b_ref, n2w_ref, n2b_ref, a2_ref, m1w_ref, m1b_ref,
                    mnw_ref, mnb_ref, ma_ref, m2w_ref, m2b_ref, out_ref,
                    smax0_s, x1_s, smax1_s, ssum_s, scnt_s):
    i = pl.program_id(0)

    @pl.when(i == 0)
    def _():
        smax0_s[...] = jnp.full((B, IN), NEG, jnp.float32)
        smax1_s[...] = jnp.full((B, HID), NEG, jnp.float32)
        ssum_s[...] = jnp.zeros((B, HID), jnp.float32)
        scnt_s[...] = jnp.zeros((B, HID), jnp.float32)

    bcol = b_ref[0]
    j = i % NB
    bmin = bnds_ref[j, 0]
    bmax = bnds_ref[j, 1]
    row = pl.multiple_of(j * RB, RB)

    @pl.when(i < NB)
    def _phase_a():
        _seg_partial_max(bmin, bmax, bcol, pos_ref[...], smax0_s)

    @pl.when((i >= NB) & (i < 2 * NB))
    def _phase_b():
        xm = _dot_t(smax0_s[...], l1w_ref[...])
        gath = _dot(_onehot(bcol), xm)
        h = _dot_t(pos_ref[...], g1w_ref[...]) + g1b_ref[...] - gath
        y = _ln_prelu(h, n1w_ref[...], n1b_ref[...], a1_ref[0, 0])
        x1_s[pl.ds(row, RB), :] = y
        _seg_partial_max(bmin, bmax, bcol, y, smax1_s)

    @pl.when(i >= 2 * NB)
    def _phase_c():
        mask = _onehot(bcol)
        xm = _dot_t(smax1_s[...], l2w_ref[...])
        gath = _dot(mask, xm)
        x = x1_s[pl.ds(row, RB), :]
        h = _dot_t(x, g2w_ref[...]) + g2b_ref[...] - gath
        y = _ln_prelu(h, n2w_ref[...], n2b_ref[...], a2_ref[0, 0])
        ssum_s[...] += jax.lax.dot_general(
            mask, y, (((0,), (0,)), ((), ())),
            preferred_element_type=jnp.float32)
        scnt_s[...] += jax.lax.dot_general(
            mask, jnp.ones((RB, HID), jnp.float32), (((0,), (0,)), ((), ())),
            preferred_element_type=jnp.float32)

        @pl.when(i == 3 * NB - 1)
        def _head():
            pooled = ssum_s[...] / jnp.maximum(scnt_s[...], 1.0)
            hh = _dot_t(pooled, m1w_ref[...]) + m1b_ref[...]
            hh = _ln_prelu(hh, mnw_ref[...], mnb_ref[...], ma_ref[0, 0])
            out_ref[...] = _dot_t(hh, m2w_ref[...]) + m2b_ref[...]


def _row(v):
    return v.reshape(1, -1)


def kernel(pos, batch, g1_w, g1_b, l1_w, n1_w, n1_b, a1, g2_w, g2_b, l2_w,
           n2_w, n2_b, a2, m1_w, m1_b, mn_w, mn_b, ma, m2_w, m2_b):
    batch = batch.astype(jnp.int32)
    b3 = batch.reshape(NB, RB, 1)
    bnds = jnp.stack([batch[::RB], batch[RB - 1::RB]], axis=1)
    a1r, a2r, mar = a1.reshape(1, 1), a2.reshape(1, 1), ma.reshape(1, 1)

    full = lambda a: pl.BlockSpec(a.shape, lambda i, b: (0,) * a.ndim)

    grid_spec = pltpu.PrefetchScalarGridSpec(
        num_scalar_prefetch=1,
        grid=(3 * NB,),
        in_specs=[
            pl.BlockSpec((RB, IN),
                         lambda i, b: (jnp.where(i < 2 * NB, i % NB, NB - 1),
                                       0)),
            pl.BlockSpec((1, RB, 1), lambda i, b: (i % NB, 0, 0)),
            full(l1_w), full(g1_w), full(_row(g1_b)), full(_row(n1_w)),
            full(_row(n1_b)), full(a1r),
            full(l2_w), full(g2_w), full(_row(g2_b)), full(_row(n2_w)),
            full(_row(n2_b)), full(a2r),
            full(m1_w), full(_row(m1_b)), full(_row(mn_w)), full(_row(mn_b)),
            full(mar), full(m2_w), full(_row(m2_b)),
        ],
        out_specs=pl.BlockSpec((B, OUT), lambda i, b: (0, 0)),
        scratch_shapes=[
            pltpu.VMEM((B, IN), jnp.float32),
            pltpu.VMEM((N, HID), jnp.float32),
            pltpu.VMEM((B, HID), jnp.float32),
            pltpu.VMEM((B, HID), jnp.float32),
            pltpu.VMEM((B, HID), jnp.float32),
        ],
    )

    out = pl.pallas_call(
        _deepset_kernel,
        grid_spec=grid_spec,
        out_shape=jax.ShapeDtypeStruct((B, OUT), jnp.float32),
    )(bnds, pos, b3, l1_w, g1_w, _row(g1_b), _row(n1_w), _row(n1_b),
      a1r, l2_w, g2_w, _row(g2_b), _row(n2_w), _row(n2_b), a2r,
      m1_w, _row(m1_b), _row(mn_w), _row(mn_b), mar, m2_w, _row(m2_b))
    return out


# speedup vs baseline: 1.2179x; 1.0309x over previous
import jax
import jax.numpy as jnp
from jax.experimental import pallas as pl
from jax.experimental.pallas import tpu as pltpu

B = 16
N = 32768
IN = 128
HID = 256
MH = HID // 2
OUT = 64
RB = 2048
NB = N // RB
NEG = -1e30
EPS = 1e-5


def _seg_partial_max(bmin, bmax, edges_ref, row, x, ref):
    io1 = jax.lax.broadcasted_iota(jnp.int32, (x.shape[0], 1), 0)

    @pl.when(bmin == bmax)
    def _():
        m = jnp.max(x, axis=0, keepdims=True)
        ref[pl.ds(bmin, 1), :] = jnp.maximum(ref[pl.ds(bmin, 1), :], m)

    @pl.when(bmin != bmax)
    def _():
        for s in range(B):
            @pl.when((bmin <= s) & (s <= bmax))
            def _(s=s):
                sel = (io1 >= edges_ref[s] - row) & (io1 < edges_ref[s + 1]
                                                     - row)
                m = jnp.max(jnp.where(sel, x, NEG), axis=0, keepdims=True)
                ref[s:s + 1, :] = jnp.maximum(ref[s:s + 1, :], m)


def _dot_t(x, w):
    return jax.lax.dot_general(
        x, w, (((1,), (1,)), ((), ())), preferred_element_type=jnp.float32
    )


def _dot(x, w):
    return jax.lax.dot_general(
        x, w, (((1,), (0,)), ((), ())), preferred_element_type=jnp.float32
    )


def _ln_prelu(h, nw, nb, a):
    mu = jnp.mean(h, axis=1, keepdims=True)
    ms = jnp.mean(h * h, axis=1, keepdims=True)
    var = jnp.maximum(ms - mu * mu, 0.0)
    y = (h - mu) * jax.lax.rsqrt(var + EPS) * nw + nb
    return jnp.where(y >= 0, y, a * y)


def _range_onehot(lo_row, hi_row, row):
    io2 = jax.lax.broadcasted_iota(jnp.int32, (RB, B), 0) + row
    return ((io2 >= lo_row) & (io2 < hi_row)).astype(jnp.float32)


def _deepset_kernel(bnds_ref, edges_ref, pos_ref, lo_ref, hi_ref, l1w_ref,
                    g1w_ref, g1b_ref, n1w_ref, n1b_ref, a1_ref, l2w_ref,
                    g2w_ref, g2b_ref, n2w_ref, n2b_ref, a2_ref, m1w_ref,
                    m1b_ref, mnw_ref, mnb_ref, ma_ref, m2w_ref, m2b_ref,
                    out_ref, smax0_s, x1_s, smax1_s, ssum_s, scnt_s):
    i = pl.program_id(0)

    @pl.when(i == 0)
    def _():
        smax0_s[...] = jnp.full((B, IN), NEG, jnp.float32)
        smax1_s[...] = jnp.full((B, HID), NEG, jnp.float32)
        ssum_s[...] = jnp.zeros((B, HID), jnp.float32)
        scnt_s[...] = jnp.zeros((B, HID), jnp.float32)

    j = i % NB
    bmin = bnds_ref[j, 0]
    bmax = bnds_ref[j, 1]
    row = pl.multiple_of(j * RB, RB)

    @pl.when(i < NB)
    def _phase_a():
        _seg_partial_max(bmin, bmax, edges_ref, row, pos_ref[...], smax0_s)

    @pl.when((i >= NB) & (i < 2 * NB))
    def _phase_b():
        xm = _dot_t(smax0_s[...], l1w_ref[...])
        gath = _dot(_range_onehot(lo_ref[...], hi_ref[...], row), xm)
        h = _dot_t(pos_ref[...], g1w_ref[...]) + g1b_ref[...] - gath
        y = _ln_prelu(h, n1w_ref[...], n1b_ref[...], a1_ref[0, 0])
        x1_s[pl.ds(row, RB), :] = y
        _seg_partial_max(bmin, bmax, edges_ref, row, y, smax1_s)

    @pl.when(i >= 2 * NB)
    def _phase_c():
        mask = _range_onehot(lo_ref[...], hi_ref[...], row)
        xm = _dot_t(smax1_s[...], l2w_ref[...])
        gath = _dot(mask, xm)
        x = x1_s[pl.ds(row, RB), :]
        h = _dot_t(x, g2w_ref[...]) + g2b_ref[...] - gath
        y = _ln_prelu(h, n2w_ref[...], n2b_ref[...], a2_ref[0, 0])
        ssum_s[...] += jax.lax.dot_general(
            mask, y, (((0,), (0,)), ((), ())),
            preferred_element_type=jnp.float32)
        scnt_s[...] += jax.lax.dot_general(
            mask, jnp.ones((RB, HID), jnp.float32), (((0,), (0,)), ((), ())),
            preferred_element_type=jnp.float32)

        @pl.when(i == 3 * NB - 1)
        def _head():
            pooled = ssum_s[...] / jnp.maximum(scnt_s[...], 1.0)
            hh = _dot_t(pooled, m1w_ref[...]) + m1b_ref[...]
            hh = _ln_prelu(hh, mnw_ref[...], mnb_ref[...], ma_ref[0, 0])
            out_ref[...] = _dot_t(hh, m2w_ref[...]) + m2b_ref[...]


def _row(v):
    return v.reshape(1, -1)


def kernel(pos, batch, g1_w, g1_b, l1_w, n1_w, n1_b, a1, g2_w, g2_b, l2_w,
           n2_w, n2_b, a2, m1_w, m1_b, mn_w, mn_b, ma, m2_w, m2_b):
    batch = batch.astype(jnp.int32)
    edges = jnp.searchsorted(batch, jnp.arange(B + 1, dtype=jnp.int32)
                             ).astype(jnp.int32)
    lo_row = edges[:B].reshape(1, B)
    hi_row = edges[1:].reshape(1, B)
    bnds = jnp.stack([batch[::RB], batch[RB - 1::RB]], axis=1)
    a1r, a2r, mar = a1.reshape(1, 1), a2.reshape(1, 1), ma.reshape(1, 1)

    full = lambda a: pl.BlockSpec(a.shape, lambda i, b, e: (0,) * a.ndim)

    grid_spec = pltpu.PrefetchScalarGridSpec(
        num_scalar_prefetch=2,
        grid=(3 * NB,),
        in_specs=[
            pl.BlockSpec((RB, IN),
                         lambda i, b, e: (jnp.where(i < 2 * NB, i % NB,
                                                    NB - 1), 0)),
            full(lo_row), full(hi_row),
            full(l1_w), full(g1_w), full(_row(g1_b)), full(_row(n1_w)),
            full(_row(n1_b)), full(a1r),
            full(l2_w), full(g2_w), full(_row(g2_b)), full(_row(n2_w)),
            full(_row(n2_b)), full(a2r),
            full(m1_w), full(_row(m1_b)), full(_row(mn_w)), full(_row(mn_b)),
            full(mar), full(m2_w), full(_row(m2_b)),
        ],
        out_specs=pl.BlockSpec((B, OUT), lambda i, b, e: (0, 0)),
        scratch_shapes=[
            pltpu.VMEM((B, IN), jnp.float32),
            pltpu.VMEM((N, HID), jnp.float32),
            pltpu.VMEM((B, HID), jnp.float32),
            pltpu.VMEM((B, HID), jnp.float32),
            pltpu.VMEM((B, HID), jnp.float32),
        ],
    )

    out = pl.pallas_call(
        _deepset_kernel,
        grid_spec=grid_spec,
        out_shape=jax.ShapeDtypeStruct((B, OUT), jnp.float32),
    )(bnds, edges, pos, lo_row, hi_row, l1_w, g1_w, _row(g1_b), _row(n1_w),
      _row(n1_b), a1r, l2_w, g2_w, _row(g2_b), _row(n2_w), _row(n2_b), a2r,
      m1_w, _row(m1_b), _row(mn_w), _row(mn_b), mar, m2_w, _row(m2_b))
    return out
